# Initial kernel scaffold; baseline (speedup 1.0000x reference)
import jax
import jax.numpy as jnp
from jax.experimental import pallas as pl


def kernel(x, edge_index, batch, W1, b1, g1, be1, rm1, rv1, W2, b2, g2, be2, rm2, rv2, W3, b3, g3, be3, rm3, rv3, Wl, bl):
    raise NotImplementedError("write your pallas kernel here")


# speedup vs baseline: 16.1654x; 16.1654x over previous
import functools

import jax
import jax.numpy as jnp
from jax import lax
from jax.experimental import pallas as pl
from jax.experimental.pallas import tpu as pltpu
from jax.experimental.pallas import tpu_sc as plsc

_N = 10000
_NPAD = 10240
_E = 320000
_NG = 16
_EPS = 1e-5

_C = 125
_NCH_TOT = _E // _C
_NW = 32
_NCHW = _NCH_TOT // _NW
_RPS = _NPAD // 16
_ZC = 128
_DEG_D = 16

_R = 2000
_GRID = _N // _R


def _fill_f32(buf, rows, d, val):
    vals = jnp.full((16,), val, jnp.float32)

    def row_body(i, _):
        def col_body(k, _):
            buf[i, pl.ds(k * 16, 16)] = vals
            return 0
        return lax.fori_loop(0, d // 16, col_body, 0)

    lax.fori_loop(0, rows, row_body, 0)


def _sc_edge_body(D, deg_mode, u_hbm, col_hbm, row_hbm, out_hbm,
                  rowi, coli, buf0, buf1, acc, g0, g1, s0, s1):
    c = lax.axis_index("c")
    s = lax.axis_index("s")
    wid = s * 2 + c

    pltpu.sync_copy(col_hbm.at[pl.ds(wid * _NCHW, _NCHW)], coli)
    if not deg_mode:
        pltpu.sync_copy(row_hbm.at[pl.ds(wid * _NCHW, _NCHW)], rowi)

    _fill_f32(buf0, _ZC, D, 0.0)
    for k in range(_RPS // _ZC):
        pltpu.sync_copy(buf0, acc.at[pl.ds(s * _RPS + k * _ZC, _ZC)])
    plsc.subcore_barrier()

    b0 = buf0.at[pl.ds(0, _C)]
    b1 = buf1.at[pl.ds(0, _C)]
    if deg_mode:
        _fill_f32(buf0, _C, D, 1.0)

        def pair(it, _):
            j = it * 2
            sd0 = pltpu.async_copy(b0, acc.at[coli.at[j]], s0, add=True)
            sd1 = pltpu.async_copy(b0, acc.at[coli.at[j + 1]], s1, add=True)
            sd0.wait()
            sd1.wait()
            return 0
    else:
        def pair(it, _):
            j = it * 2
            gd0 = pltpu.async_copy(u_hbm.at[rowi.at[j]], b0, g0)
            gd1 = pltpu.async_copy(u_hbm.at[rowi.at[j + 1]], b1, g1)
            gd0.wait()
            sd0 = pltpu.async_copy(b0, acc.at[coli.at[j]], s0, add=True)
            gd1.wait()
            sd1 = pltpu.async_copy(b1, acc.at[coli.at[j + 1]], s1, add=True)
            sd0.wait()
            sd1.wait()
            return 0

    lax.fori_loop(0, _NCHW // 2, pair, 0)
    plsc.subcore_barrier()

    sl = pl.ds(s * _RPS, _RPS)
    pltpu.sync_copy(acc.at[sl], out_hbm.at[c].at[sl])


def _sc_scratch(D):
    return [
        pltpu.VMEM((_NCHW, _C), jnp.int32),
        pltpu.VMEM((_NCHW, _C), jnp.int32),
        pltpu.VMEM((_ZC, D), jnp.float32),
        pltpu.VMEM((_ZC, D), jnp.float32),
        pltpu.VMEM_SHARED((_NPAD, D), jnp.float32),
        pltpu.SemaphoreType.DMA,
        pltpu.SemaphoreType.DMA,
        pltpu.SemaphoreType.DMA,
        pltpu.SemaphoreType.DMA,
    ]


@functools.cache
def _make_sc_pass(D):
    mesh = plsc.VectorSubcoreMesh(core_axis_name="c", subcore_axis_name="s")

    @functools.partial(
        pl.kernel, mesh=mesh,
        out_type=jax.ShapeDtypeStruct((2, _NPAD, D), jnp.float32),
        scratch_types=_sc_scratch(D),
        compiler_params=pltpu.CompilerParams(use_tc_tiling_on_sc=False),
        name=f"sc_edge_pass_{D}",
    )
    def sc_pass(u_hbm, row_hbm, col_hbm, out_hbm,
                rowi, coli, buf0, buf1, acc, g0, g1, s0, s1):
        _sc_edge_body(D, False, u_hbm, col_hbm, row_hbm, out_hbm,
                      rowi, coli, buf0, buf1, acc, g0, g1, s0, s1)

    return sc_pass


@functools.cache
def _make_sc_deg():
    mesh = plsc.VectorSubcoreMesh(core_axis_name="c", subcore_axis_name="s")
    D = _DEG_D

    @functools.partial(
        pl.kernel, mesh=mesh,
        out_type=jax.ShapeDtypeStruct((2, _NPAD, D), jnp.float32),
        scratch_types=_sc_scratch(D),
        compiler_params=pltpu.CompilerParams(use_tc_tiling_on_sc=False),
        name="sc_deg_pass",
    )
    def sc_deg(row_hbm, out_hbm,
               rowi, coli, buf0, buf1, acc, g0, g1, s0, s1):
        _sc_edge_body(D, True, None, row_hbm, None, out_hbm,
                      rowi, coli, buf0, buf1, acc, g0, g1, s0, s1)

    return sc_deg


def _lrelu(z):
    return jnp.where(z >= 0, z, 0.01 * z)


def _tc_start(x, wcat, degp, d_next, d_out):
    def body(x_ref, w_ref, d_ref, ua_ref, ub_ref, base_ref, dis_ref):
        deg = d_ref[0, :, 0:1] + d_ref[1, :, 0:1]
        dis = jnp.where(deg > 0, lax.rsqrt(deg), 0.0)
        mm = jnp.dot(x_ref[...], w_ref[...], preferred_element_type=jnp.float32)
        ua_ref[...] = dis * mm[:, :d_next]
        ub_ref[...] = dis * mm[:, d_next:2 * d_next]
        base_ref[...] = mm[:, 2 * d_next:]
        dis_ref[...] = dis

    f_in = x.shape[1]
    return pl.pallas_call(
        body,
        grid=(_GRID,),
        in_specs=[
            pl.BlockSpec((_R, f_in), lambda i: (i, 0)),
            pl.BlockSpec((f_in, 2 * d_next + d_out), lambda i: (0, 0)),
            pl.BlockSpec((2, _R, _DEG_D), lambda i: (0, i, 0)),
        ],
        out_specs=[
            pl.BlockSpec((_R, d_next), lambda i: (i, 0)),
            pl.BlockSpec((_R, d_next), lambda i: (i, 0)),
            pl.BlockSpec((_R, d_out), lambda i: (i, 0)),
            pl.BlockSpec((_R, 1), lambda i: (i, 0)),
        ],
        out_shape=[
            jax.ShapeDtypeStruct((_N, d_next), jnp.float32),
            jax.ShapeDtypeStruct((_N, d_next), jnp.float32),
            jax.ShapeDtypeStruct((_N, d_out), jnp.float32),
            jax.ShapeDtypeStruct((_N, 1), jnp.float32),
        ],
    )(x, wcat, degp)


def _tc_mid(pa, pb, dis, d):
    def body(pa_ref, pb_ref, dis_ref, t1_ref, u2_ref):
        dis_v = dis_ref[...]
        t1_ref[...] = -dis_v * (pa_ref[0] + pa_ref[1])
        u2_ref[...] = (dis_v * dis_v) * (pb_ref[0] + pb_ref[1])

    return pl.pallas_call(
        body,
        grid=(_GRID,),
        in_specs=[
            pl.BlockSpec((2, _R, d), lambda i: (0, i, 0)),
            pl.BlockSpec((2, _R, d), lambda i: (0, i, 0)),
            pl.BlockSpec((_R, 1), lambda i: (i, 0)),
        ],
        out_specs=[
            pl.BlockSpec((_R, d), lambda i: (i, 0)),
            pl.BlockSpec((_R, d), lambda i: (i, 0)),
        ],
        out_shape=[
            jax.ShapeDtypeStruct((_N, d), jnp.float32),
            jax.ShapeDtypeStruct((_N, d), jnp.float32),
        ],
    )(pa, pb, dis)


def _tc_end(base, t1, q, dis, scale, shift, wcat, d, dn_next, dn_out):
    def body(b_ref, t1_ref, q_ref, dis_ref, sc_ref, sh_ref, w_ref,
             ua_ref, ub_ref, base_ref):
        dis_v = dis_ref[...]
        cheb = b_ref[...] + t1_ref[...] + 2.0 * dis_v * (q_ref[0] + q_ref[1])
        h = _lrelu(cheb * sc_ref[...] + sh_ref[...])
        mm = jnp.dot(h, w_ref[...], preferred_element_type=jnp.float32)
        ua_ref[...] = dis_v * mm[:, :dn_next]
        ub_ref[...] = dis_v * mm[:, dn_next:2 * dn_next]
        base_ref[...] = mm[:, 2 * dn_next:]

    return pl.pallas_call(
        body,
        grid=(_GRID,),
        in_specs=[
            pl.BlockSpec((_R, d), lambda i: (i, 0)),
            pl.BlockSpec((_R, d), lambda i: (i, 0)),
            pl.BlockSpec((2, _R, d), lambda i: (0, i, 0)),
            pl.BlockSpec((_R, 1), lambda i: (i, 0)),
            pl.BlockSpec((1, d), lambda i: (0, 0)),
            pl.BlockSpec((1, d), lambda i: (0, 0)),
            pl.BlockSpec((d, 2 * dn_next + dn_out), lambda i: (0, 0)),
        ],
        out_specs=[
            pl.BlockSpec((_R, dn_next), lambda i: (i, 0)),
            pl.BlockSpec((_R, dn_next), lambda i: (i, 0)),
            pl.BlockSpec((_R, dn_out), lambda i: (i, 0)),
        ],
        out_shape=[
            jax.ShapeDtypeStruct((_N, dn_next), jnp.float32),
            jax.ShapeDtypeStruct((_N, dn_next), jnp.float32),
            jax.ShapeDtypeStruct((_N, dn_out), jnp.float32),
        ],
    )(base, t1, q, dis, scale, shift, wcat)


def _tc_final(base, t1, q, dis, scale, shift, batch2d, wl, bl, d):
    def body(b_ref, t1_ref, q_ref, dis_ref, sc_ref, sh_ref,
             bt_ref, wl_ref, bl_ref, out_ref, sums, counts):
        i = pl.program_id(0)

        @pl.when(i == 0)
        def _():
            sums[...] = jnp.zeros_like(sums)
            counts[...] = jnp.zeros_like(counts)

        dis_v = dis_ref[...]
        cheb = b_ref[...] + t1_ref[...] + 2.0 * dis_v * (q_ref[0] + q_ref[1])
        h = _lrelu(cheb * sc_ref[...] + sh_ref[...])
        gids = lax.broadcasted_iota(jnp.int32, (1, _NG), 1)
        mask = (bt_ref[...] == gids).astype(jnp.float32)
        sums[...] += lax.dot_general(
            mask, h, (((0,), (0,)), ((), ())),
            preferred_element_type=jnp.float32)
        ones_col = jnp.ones((_R, 1), jnp.float32)
        counts[...] += lax.dot_general(
            mask, ones_col, (((0,), (0,)), ((), ())),
            preferred_element_type=jnp.float32)

        @pl.when(i == _GRID - 1)
        def _():
            mean = sums[...] / jnp.maximum(counts[...], 1.0)
            out_ref[...] = jnp.dot(
                mean, wl_ref[...], preferred_element_type=jnp.float32
            ) + bl_ref[...]

    return pl.pallas_call(
        body,
        grid=(_GRID,),
        in_specs=[
            pl.BlockSpec((_R, d), lambda i: (i, 0)),
            pl.BlockSpec((_R, d), lambda i: (i, 0)),
            pl.BlockSpec((2, _R, d), lambda i: (0, i, 0)),
            pl.BlockSpec((_R, 1), lambda i: (i, 0)),
            pl.BlockSpec((1, d), lambda i: (0, 0)),
            pl.BlockSpec((1, d), lambda i: (0, 0)),
            pl.BlockSpec((_R, 1), lambda i: (i, 0)),
            pl.BlockSpec((d, 2), lambda i: (0, 0)),
            pl.BlockSpec((1, 2), lambda i: (0, 0)),
        ],
        out_specs=pl.BlockSpec((_NG, 2), lambda i: (0, 0)),
        out_shape=jax.ShapeDtypeStruct((_NG, 2), jnp.float32),
        scratch_shapes=[
            pltpu.VMEM((_NG, 16), jnp.float32),
            pltpu.VMEM((_NG, 1), jnp.float32),
        ],
    )(base, t1, q, dis, scale, shift, batch2d, wl, bl)


def _bn_fold(b, g, be, rm, rv):
    s = g / jnp.sqrt(rv + _EPS)
    return s[None, :], (b * s + be - rm * s)[None, :]


@jax.jit
def kernel(x, edge_index, batch,
           W1, b1, g1, be1, rm1, rv1,
           W2, b2, g2, be2, rm2, rv2,
           W3, b3, g3, be3, rm3, rv3,
           Wl, bl):
    row2d = edge_index[0].reshape(_NCH_TOT, _C)
    col2d = edge_index[1].reshape(_NCH_TOT, _C)

    wcat1 = jnp.concatenate([W1[1], W1[2], W1[0] - W1[2]], axis=1)
    wcat2 = jnp.concatenate([W2[1], W2[2], W2[0] - W2[2]], axis=1)
    wcat3 = jnp.concatenate([W3[1], W3[2], W3[0] - W3[2]], axis=1)
    sc1, sh1 = _bn_fold(b1, g1, be1, rm1, rv1)
    sc2, sh2 = _bn_fold(b2, g2, be2, rm2, rv2)
    sc3, sh3 = _bn_fold(b3, g3, be3, rm3, rv3)

    degp = _make_sc_deg()(row2d)
    ua, ub, base, dis = _tc_start(x, wcat1, degp, 64, 64)

    pa = _make_sc_pass(64)(ua, row2d, col2d)
    pb = _make_sc_pass(64)(ub, row2d, col2d)
    t1, u2 = _tc_mid(pa, pb, dis, 64)
    q = _make_sc_pass(64)(u2, row2d, col2d)
    ua, ub, base = _tc_end(base, t1, q, dis, sc1, sh1, wcat2, 64, 32, 32)

    pa = _make_sc_pass(32)(ua, row2d, col2d)
    pb = _make_sc_pass(32)(ub, row2d, col2d)
    t1, u2 = _tc_mid(pa, pb, dis, 32)
    q = _make_sc_pass(32)(u2, row2d, col2d)
    ua, ub, base = _tc_end(base, t1, q, dis, sc2, sh2, wcat3, 32, 16, 16)

    pa = _make_sc_pass(16)(ua, row2d, col2d)
    pb = _make_sc_pass(16)(ub, row2d, col2d)
    t1, u2 = _tc_mid(pa, pb, dis, 16)
    q = _make_sc_pass(16)(u2, row2d, col2d)

    batch2d = batch.reshape(_N, 1)
    return _tc_final(base, t1, q, dis, sc3, sh3, batch2d,
                     Wl, bl[None, :], 16)


# speedup vs baseline: 21.6917x; 1.3419x over previous
import functools

import jax
import jax.numpy as jnp
from jax import lax
from jax.experimental import pallas as pl
from jax.experimental.pallas import tpu as pltpu
from jax.experimental.pallas import tpu_sc as plsc

_N = 10000
_NPAD = 10240
_E = 320000
_NG = 16
_EPS = 1e-5

_C = 125
_NCH_TOT = _E // _C
_NW = 32
_NCHW = _NCH_TOT // _NW
_RPS = _NPAD // 16
_ZC = 128
_DEG_D = 16

_R = 2000
_GRID = _N // _R


def _fill_f32(buf, rows, d, val):
    vals = jnp.full((16,), val, jnp.float32)

    def row_body(i, _):
        def col_body(k, _):
            buf[i, pl.ds(k * 16, 16)] = vals
            return 0
        return lax.fori_loop(0, d // 16, col_body, 0)

    lax.fori_loop(0, rows, row_body, 0)


_NBUF = 5
_NGRP = _NCHW // _NBUF


def _sc_edge_body(D, deg_mode, u_hbm, col_hbm, row_hbm, out_hbm, refs):
    rowi, coli = refs[0], refs[1]
    bufs = refs[2:2 + _NBUF]
    acc = refs[2 + _NBUF]
    gsems = refs[3 + _NBUF:3 + 2 * _NBUF]
    ssems = refs[3 + 2 * _NBUF:3 + 3 * _NBUF]

    c = lax.axis_index("c")
    s = lax.axis_index("s")
    wid = s * 2 + c

    pltpu.sync_copy(col_hbm.at[pl.ds(wid * _NCHW, _NCHW)], coli)
    if not deg_mode:
        pltpu.sync_copy(row_hbm.at[pl.ds(wid * _NCHW, _NCHW)], rowi)

    _fill_f32(bufs[0], _ZC, D, 0.0)
    for k in range(_RPS // _ZC):
        pltpu.sync_copy(bufs[0], acc.at[pl.ds(s * _RPS + k * _ZC, _ZC)])
    plsc.subcore_barrier()

    b0s = [buf.at[pl.ds(0, _C)] for buf in bufs]

    if deg_mode:
        _fill_f32(bufs[0], _C, D, 1.0)

        def grp(it, _):
            base = it * _NBUF
            for b in range(_NBUF):
                pltpu.async_copy(b0s[0], acc.at[coli.at[base + b]],
                                 ssems[b], add=True)
            for b in range(_NBUF):
                pltpu.make_async_copy(
                    b0s[0], acc.at[coli.at[0]], ssems[b]).wait()
            return 0
    else:
        for b in range(_NBUF):
            pltpu.async_copy(u_hbm.at[rowi.at[b]], b0s[b], gsems[b])

        def grp(it, _):
            base = it * _NBUF
            for b in range(_NBUF):
                pltpu.make_async_copy(
                    u_hbm.at[rowi.at[0]], b0s[b], gsems[b]).wait()
                pltpu.async_copy(b0s[b], acc.at[coli.at[base + b]],
                                 ssems[b], add=True)
            nxt = base + _NBUF
            for b in range(_NBUF):
                pltpu.make_async_copy(
                    b0s[b], acc.at[coli.at[0]], ssems[b]).wait()

                @pl.when(nxt + b < _NCHW)
                def _():
                    pltpu.async_copy(u_hbm.at[rowi.at[nxt + b]],
                                     b0s[b], gsems[b])
            return 0

    lax.fori_loop(0, _NGRP, grp, 0)
    plsc.subcore_barrier()

    sl = pl.ds(s * _RPS, _RPS)
    pltpu.sync_copy(acc.at[sl], out_hbm.at[c].at[sl])


def _sc_scratch(D):
    return ([
        pltpu.VMEM((_NCHW, _C), jnp.int32),
        pltpu.VMEM((_NCHW, _C), jnp.int32),
    ] + [pltpu.VMEM((_ZC, D), jnp.float32)] * _NBUF
      + [pltpu.VMEM_SHARED((_NPAD, D), jnp.float32)]
      + [pltpu.SemaphoreType.DMA] * (2 * _NBUF))


@functools.cache
def _make_sc_pass(D):
    mesh = plsc.VectorSubcoreMesh(core_axis_name="c", subcore_axis_name="s")

    @functools.partial(
        pl.kernel, mesh=mesh,
        out_type=jax.ShapeDtypeStruct((2, _NPAD, D), jnp.float32),
        scratch_types=_sc_scratch(D),
        compiler_params=pltpu.CompilerParams(use_tc_tiling_on_sc=False),
        name=f"sc_edge_pass_{D}",
    )
    def sc_pass(u_hbm, row_hbm, col_hbm, out_hbm, *refs):
        _sc_edge_body(D, False, u_hbm, col_hbm, row_hbm, out_hbm, refs)

    return sc_pass


@functools.cache
def _make_sc_deg():
    mesh = plsc.VectorSubcoreMesh(core_axis_name="c", subcore_axis_name="s")
    D = _DEG_D

    @functools.partial(
        pl.kernel, mesh=mesh,
        out_type=jax.ShapeDtypeStruct((2, _NPAD, D), jnp.float32),
        scratch_types=_sc_scratch(D),
        compiler_params=pltpu.CompilerParams(use_tc_tiling_on_sc=False),
        name="sc_deg_pass",
    )
    def sc_deg(row_hbm, out_hbm, *refs):
        _sc_edge_body(D, True, None, row_hbm, None, out_hbm, refs)

    return sc_deg


def _lrelu(z):
    return jnp.where(z >= 0, z, 0.01 * z)


def _tc_start(x, wcat, degp, d_next, d_out):
    def body(x_ref, w_ref, d_ref, ua_ref, ub_ref, base_ref, dis_ref):
        deg = d_ref[0, :, 0:1] + d_ref[1, :, 0:1]
        dis = jnp.where(deg > 0, lax.rsqrt(deg), 0.0)
        mm = jnp.dot(x_ref[...], w_ref[...], preferred_element_type=jnp.float32)
        ua_ref[...] = dis * mm[:, :d_next]
        ub_ref[...] = dis * mm[:, d_next:2 * d_next]
        base_ref[...] = mm[:, 2 * d_next:]
        dis_ref[...] = dis

    f_in = x.shape[1]
    return pl.pallas_call(
        body,
        grid=(_GRID,),
        in_specs=[
            pl.BlockSpec((_R, f_in), lambda i: (i, 0)),
            pl.BlockSpec((f_in, 2 * d_next + d_out), lambda i: (0, 0)),
            pl.BlockSpec((2, _R, _DEG_D), lambda i: (0, i, 0)),
        ],
        out_specs=[
            pl.BlockSpec((_R, d_next), lambda i: (i, 0)),
            pl.BlockSpec((_R, d_next), lambda i: (i, 0)),
            pl.BlockSpec((_R, d_out), lambda i: (i, 0)),
            pl.BlockSpec((_R, 1), lambda i: (i, 0)),
        ],
        out_shape=[
            jax.ShapeDtypeStruct((_N, d_next), jnp.float32),
            jax.ShapeDtypeStruct((_N, d_next), jnp.float32),
            jax.ShapeDtypeStruct((_N, d_out), jnp.float32),
            jax.ShapeDtypeStruct((_N, 1), jnp.float32),
        ],
    )(x, wcat, degp)


def _tc_mid(pa, pb, dis, d):
    def body(pa_ref, pb_ref, dis_ref, t1_ref, u2_ref):
        dis_v = dis_ref[...]
        t1_ref[...] = -dis_v * (pa_ref[0] + pa_ref[1])
        u2_ref[...] = (dis_v * dis_v) * (pb_ref[0] + pb_ref[1])

    return pl.pallas_call(
        body,
        grid=(_GRID,),
        in_specs=[
            pl.BlockSpec((2, _R, d), lambda i: (0, i, 0)),
            pl.BlockSpec((2, _R, d), lambda i: (0, i, 0)),
            pl.BlockSpec((_R, 1), lambda i: (i, 0)),
        ],
        out_specs=[
            pl.BlockSpec((_R, d), lambda i: (i, 0)),
            pl.BlockSpec((_R, d), lambda i: (i, 0)),
        ],
        out_shape=[
            jax.ShapeDtypeStruct((_N, d), jnp.float32),
            jax.ShapeDtypeStruct((_N, d), jnp.float32),
        ],
    )(pa, pb, dis)


def _tc_end(base, t1, q, dis, scale, shift, wcat, d, dn_next, dn_out):
    def body(b_ref, t1_ref, q_ref, dis_ref, sc_ref, sh_ref, w_ref,
             ua_ref, ub_ref, base_ref):
        dis_v = dis_ref[...]
        cheb = b_ref[...] + t1_ref[...] + 2.0 * dis_v * (q_ref[0] + q_ref[1])
        h = _lrelu(cheb * sc_ref[...] + sh_ref[...])
        mm = jnp.dot(h, w_ref[...], preferred_element_type=jnp.float32)
        ua_ref[...] = dis_v * mm[:, :dn_next]
        ub_ref[...] = dis_v * mm[:, dn_next:2 * dn_next]
        base_ref[...] = mm[:, 2 * dn_next:]

    return pl.pallas_call(
        body,
        grid=(_GRID,),
        in_specs=[
            pl.BlockSpec((_R, d), lambda i: (i, 0)),
            pl.BlockSpec((_R, d), lambda i: (i, 0)),
            pl.BlockSpec((2, _R, d), lambda i: (0, i, 0)),
            pl.BlockSpec((_R, 1), lambda i: (i, 0)),
            pl.BlockSpec((1, d), lambda i: (0, 0)),
            pl.BlockSpec((1, d), lambda i: (0, 0)),
            pl.BlockSpec((d, 2 * dn_next + dn_out), lambda i: (0, 0)),
        ],
        out_specs=[
            pl.BlockSpec((_R, dn_next), lambda i: (i, 0)),
            pl.BlockSpec((_R, dn_next), lambda i: (i, 0)),
            pl.BlockSpec((_R, dn_out), lambda i: (i, 0)),
        ],
        out_shape=[
            jax.ShapeDtypeStruct((_N, dn_next), jnp.float32),
            jax.ShapeDtypeStruct((_N, dn_next), jnp.float32),
            jax.ShapeDtypeStruct((_N, dn_out), jnp.float32),
        ],
    )(base, t1, q, dis, scale, shift, wcat)


def _tc_final(base, t1, q, dis, scale, shift, batch2d, wl, bl, d):
    def body(b_ref, t1_ref, q_ref, dis_ref, sc_ref, sh_ref,
             bt_ref, wl_ref, bl_ref, out_ref, sums, counts):
        i = pl.program_id(0)

        @pl.when(i == 0)
        def _():
            sums[...] = jnp.zeros_like(sums)
            counts[...] = jnp.zeros_like(counts)

        dis_v = dis_ref[...]
        cheb = b_ref[...] + t1_ref[...] + 2.0 * dis_v * (q_ref[0] + q_ref[1])
        h = _lrelu(cheb * sc_ref[...] + sh_ref[...])
        gids = lax.broadcasted_iota(jnp.int32, (1, _NG), 1)
        mask = (bt_ref[...] == gids).astype(jnp.float32)
        sums[...] += lax.dot_general(
            mask, h, (((0,), (0,)), ((), ())),
            preferred_element_type=jnp.float32)
        ones_col = jnp.ones((_R, 1), jnp.float32)
        counts[...] += lax.dot_general(
            mask, ones_col, (((0,), (0,)), ((), ())),
            preferred_element_type=jnp.float32)

        @pl.when(i == _GRID - 1)
        def _():
            mean = sums[...] / jnp.maximum(counts[...], 1.0)
            out_ref[...] = jnp.dot(
                mean, wl_ref[...], preferred_element_type=jnp.float32
            ) + bl_ref[...]

    return pl.pallas_call(
        body,
        grid=(_GRID,),
        in_specs=[
            pl.BlockSpec((_R, d), lambda i: (i, 0)),
            pl.BlockSpec((_R, d), lambda i: (i, 0)),
            pl.BlockSpec((2, _R, d), lambda i: (0, i, 0)),
            pl.BlockSpec((_R, 1), lambda i: (i, 0)),
            pl.BlockSpec((1, d), lambda i: (0, 0)),
            pl.BlockSpec((1, d), lambda i: (0, 0)),
            pl.BlockSpec((_R, 1), lambda i: (i, 0)),
            pl.BlockSpec((d, 2), lambda i: (0, 0)),
            pl.BlockSpec((1, 2), lambda i: (0, 0)),
        ],
        out_specs=pl.BlockSpec((_NG, 2), lambda i: (0, 0)),
        out_shape=jax.ShapeDtypeStruct((_NG, 2), jnp.float32),
        scratch_shapes=[
            pltpu.VMEM((_NG, 16), jnp.float32),
            pltpu.VMEM((_NG, 1), jnp.float32),
        ],
    )(base, t1, q, dis, scale, shift, batch2d, wl, bl)


def _bn_fold(b, g, be, rm, rv):
    s = g / jnp.sqrt(rv + _EPS)
    return s[None, :], (b * s + be - rm * s)[None, :]


@jax.jit
def kernel(x, edge_index, batch,
           W1, b1, g1, be1, rm1, rv1,
           W2, b2, g2, be2, rm2, rv2,
           W3, b3, g3, be3, rm3, rv3,
           Wl, bl):
    row2d = edge_index[0].reshape(_NCH_TOT, _C)
    col2d = edge_index[1].reshape(_NCH_TOT, _C)

    wcat1 = jnp.concatenate([W1[1], W1[2], W1[0] - W1[2]], axis=1)
    wcat2 = jnp.concatenate([W2[1], W2[2], W2[0] - W2[2]], axis=1)
    wcat3 = jnp.concatenate([W3[1], W3[2], W3[0] - W3[2]], axis=1)
    sc1, sh1 = _bn_fold(b1, g1, be1, rm1, rv1)
    sc2, sh2 = _bn_fold(b2, g2, be2, rm2, rv2)
    sc3, sh3 = _bn_fold(b3, g3, be3, rm3, rv3)

    degp = _make_sc_deg()(row2d)
    ua, ub, base, dis = _tc_start(x, wcat1, degp, 64, 64)

    pa = _make_sc_pass(64)(ua, row2d, col2d)
    pb = _make_sc_pass(64)(ub, row2d, col2d)
    t1, u2 = _tc_mid(pa, pb, dis, 64)
    q = _make_sc_pass(64)(u2, row2d, col2d)
    ua, ub, base = _tc_end(base, t1, q, dis, sc1, sh1, wcat2, 64, 32, 32)

    pa = _make_sc_pass(32)(ua, row2d, col2d)
    pb = _make_sc_pass(32)(ub, row2d, col2d)
    t1, u2 = _tc_mid(pa, pb, dis, 32)
    q = _make_sc_pass(32)(u2, row2d, col2d)
    ua, ub, base = _tc_end(base, t1, q, dis, sc2, sh2, wcat3, 32, 16, 16)

    pa = _make_sc_pass(16)(ua, row2d, col2d)
    pb = _make_sc_pass(16)(ub, row2d, col2d)
    t1, u2 = _tc_mid(pa, pb, dis, 16)
    q = _make_sc_pass(16)(u2, row2d, col2d)

    batch2d = batch.reshape(_N, 1)
    return _tc_final(base, t1, q, dis, sc3, sh3, batch2d,
                     Wl, bl[None, :], 16)
